# Initial kernel scaffold; baseline (speedup 1.0000x reference)
import jax
import jax.numpy as jnp
from jax.experimental import pallas as pl


def kernel(x, W):
    raise NotImplementedError("write your pallas kernel here")


# speedup vs baseline: 3.1672x; 3.1672x over previous
import functools

import jax
import jax.numpy as jnp
from jax import lax
from jax.experimental import pallas as pl
from jax.experimental.pallas import tpu as pltpu
from jax.experimental.pallas import tpu_sc as plsc

_NC = 2
_NS = 16
_NW = _NC * _NS

_IDX_W = 128
_ROWS_PER_CHUNK = 4
_CHUNK = _IDX_W * _ROWS_PER_CHUNK


@functools.partial(jax.jit, static_argnames=("n2", "dim2"))
def _sc_lookup(W2, idx2d, *, n2, dim2):
    per_w = n2 // _NW
    chunks = per_w // _CHUNK
    rows_per_w = per_w // _IDX_W

    mesh = plsc.VectorSubcoreMesh(core_axis_name="c", subcore_axis_name="s")

    @functools.partial(
        pl.kernel,
        mesh=mesh,
        out_type=jax.ShapeDtypeStruct((n2, dim2), jnp.float32),
        scratch_types=[
            pltpu.VMEM((_ROWS_PER_CHUNK, _IDX_W), jnp.int32),
            pltpu.VMEM((_CHUNK, dim2), jnp.float32),
            pltpu.SemaphoreType.DMA,
        ],
    )
    def body(table_hbm, idx_hbm, out_hbm, idx_v, rows_v, sem):
        wid = lax.axis_index("s") * _NC + lax.axis_index("c")
        row0 = wid * rows_per_w

        def step(g, carry):
            r = row0 + g * _ROWS_PER_CHUNK
            pltpu.sync_copy(idx_hbm.at[pl.ds(r, _ROWS_PER_CHUNK)], idx_v)
            for j in range(_ROWS_PER_CHUNK):
                pltpu.async_copy(
                    table_hbm.at[idx_v.at[j]],
                    rows_v.at[pl.ds(j * _IDX_W, _IDX_W)],
                    sem,
                )
            for j in range(_ROWS_PER_CHUNK):
                pltpu.make_async_copy(
                    table_hbm.at[idx_v.at[j]],
                    rows_v.at[pl.ds(j * _IDX_W, _IDX_W)],
                    sem,
                ).wait()
            pltpu.sync_copy(rows_v, out_hbm.at[pl.ds(r * _IDX_W, _CHUNK)])
            return carry

        lax.fori_loop(0, chunks, step, 0)

    return body(W2, idx2d)


def kernel(x, W):
    b0, b1 = x.shape
    vocab, dim = W.shape
    n = b0 * b1
    n2 = n // 2
    W2 = jnp.concatenate(
        [jnp.repeat(W, vocab, axis=0), jnp.tile(W, (vocab, 1))], axis=1
    )
    xp = x.reshape(n2, 2).astype(jnp.int32)
    idx2 = xp[:, 0] * vocab + xp[:, 1]
    idx2d = idx2.reshape(n2 // _IDX_W, _IDX_W)
    out = _sc_lookup(W2, idx2d, n2=n2, dim2=2 * dim)
    return out.reshape(b0, b1, dim)


# speedup vs baseline: 3.1824x; 1.0048x over previous
import functools

import jax
import jax.numpy as jnp
from jax import lax
from jax.experimental import pallas as pl
from jax.experimental.pallas import tpu as pltpu
from jax.experimental.pallas import tpu_sc as plsc

_NC = 2
_NS = 16
_NW = _NC * _NS

_IDX_W = 128
_ROWS_PER_CHUNK = 2
_CHUNK = _IDX_W * _ROWS_PER_CHUNK
_NBUF = 2


@functools.partial(jax.jit, static_argnames=("n2", "dim2"))
def _sc_lookup(W2, idx2d, *, n2, dim2):
    per_w = n2 // _NW
    chunks = per_w // _CHUNK
    rows_per_w = per_w // _IDX_W
    assert chunks % _NBUF == 0

    mesh = plsc.VectorSubcoreMesh(core_axis_name="c", subcore_axis_name="s")

    @functools.partial(
        pl.kernel,
        mesh=mesh,
        out_type=jax.ShapeDtypeStruct((n2, dim2), jnp.float32),
        scratch_types=[
            pltpu.VMEM((_NBUF, _ROWS_PER_CHUNK, _IDX_W), jnp.int32),
            pltpu.VMEM((_NBUF, _CHUNK, dim2), jnp.float32),
            pltpu.SemaphoreType.DMA((_NBUF,)),
            pltpu.SemaphoreType.DMA((_NBUF,)),
            pltpu.SemaphoreType.DMA((_NBUF,)),
        ],
    )
    def body(table_hbm, idx_hbm, out_hbm, idx_v, rows_v, isem, gsem, ssem):
        wid = lax.axis_index("s") * _NC + lax.axis_index("c")
        row0 = wid * rows_per_w

        def idx_load(g, b):
            return pltpu.make_async_copy(
                idx_hbm.at[pl.ds(row0 + g * _ROWS_PER_CHUNK, _ROWS_PER_CHUNK)],
                idx_v.at[b],
                isem.at[b],
            )

        def gather(g, b, j):
            return pltpu.make_async_copy(
                table_hbm.at[idx_v.at[b, j]],
                rows_v.at[b, pl.ds(j * _IDX_W, _IDX_W)],
                gsem.at[b],
            )

        def store(g, b):
            return pltpu.make_async_copy(
                rows_v.at[b],
                out_hbm.at[pl.ds((row0 + g * _ROWS_PER_CHUNK) * _IDX_W, _CHUNK)],
                ssem.at[b],
            )

        for b in range(_NBUF):
            idx_load(b, b).start()

        def outer(i, carry):
            g0 = i * _NBUF
            for b in range(_NBUF):
                g = g0 + b
                idx_load(g, b).wait()
                @pl.when(g0 > 0)
                def _():
                    store(g, b).wait()

                for j in range(_ROWS_PER_CHUNK):
                    gather(g, b, j).start()
                for j in range(_ROWS_PER_CHUNK):
                    gather(g, b, j).wait()

                @pl.when(g0 < chunks - _NBUF)
                def _():
                    idx_load(g + _NBUF, b).start()

                store(g, b).start()
            return carry

        lax.fori_loop(0, chunks // _NBUF, outer, 0)
        for b in range(_NBUF):
            store(chunks - _NBUF + b, b).wait()

    return body(W2, idx2d)


def kernel(x, W):
    b0, b1 = x.shape
    vocab, dim = W.shape
    n = b0 * b1
    n2 = n // 2
    W2 = jnp.concatenate(
        [jnp.repeat(W, vocab, axis=0), jnp.tile(W, (vocab, 1))], axis=1
    )
    xp = x.reshape(n2, 2).astype(jnp.int32)
    idx2 = xp[:, 0] * vocab + xp[:, 1]
    idx2d = idx2.reshape(n2 // _IDX_W, _IDX_W)
    out = _sc_lookup(W2, idx2d, n2=n2, dim2=2 * dim)
    return out.reshape(b0, b1, dim)
